# baseline (speedup 1.0000x reference)
import jax
import jax.numpy as jnp
from jax.experimental import pallas as pl
from jax.experimental.pallas import tpu as pltpu

N, D, H, K = 8192, 1024, 16384, 64
NB, HB = 2048, 1024
NB2, HB2 = 2048, 512


def _mm_kernel(x_ref, w_ref, b_ref, o_ref):
    o_ref[...] = jax.lax.dot_general(
        x_ref[...], w_ref[...], (((1,), (1,)), ((), ())),
        preferred_element_type=jnp.float32) + b_ref[...]


def _encode(xc, W_enc, latent_bias):
    return pl.pallas_call(
        _mm_kernel,
        grid=(N // NB, H // HB),
        in_specs=[
            pl.BlockSpec((NB, D), lambda i, j: (i, 0)),
            pl.BlockSpec((HB, D), lambda i, j: (j, 0)),
            pl.BlockSpec((1, HB), lambda i, j: (0, j)),
        ],
        out_specs=pl.BlockSpec((NB, HB), lambda i, j: (i, j)),
        out_shape=jax.ShapeDtypeStruct((N, H), jnp.float32),
    )(xc, W_enc, latent_bias.reshape(1, H))


def _scatter_decode_kernel(pa_ref, t_ref, wdt_ref, sc_ref, rec_ref):
    j = pl.program_id(1)
    relu = jnp.maximum(pa_ref[...], 0.0)
    sc = jnp.where(relu >= t_ref[...], relu, 0.0)
    sc_ref[...] = sc

    @pl.when(j == 0)
    def _():
        rec_ref[...] = jnp.zeros_like(rec_ref)

    rec_ref[...] += jax.lax.dot_general(
        sc, wdt_ref[...], (((1,), (0,)), ((), ())),
        preferred_element_type=jnp.float32)


def _scatter_decode(pre_acts, thresh, WdT):
    return pl.pallas_call(
        _scatter_decode_kernel,
        grid=(N // NB2, H // HB2),
        in_specs=[
            pl.BlockSpec((NB2, HB2), lambda i, j: (i, j)),
            pl.BlockSpec((NB2, 1), lambda i, j: (i, 0)),
            pl.BlockSpec((HB2, D), lambda i, j: (j, 0)),
        ],
        out_specs=[
            pl.BlockSpec((NB2, HB2), lambda i, j: (i, j)),
            pl.BlockSpec((NB2, D), lambda i, j: (i, 0)),
        ],
        out_shape=[
            jax.ShapeDtypeStruct((N, H), jnp.float32),
            jax.ShapeDtypeStruct((N, D), jnp.float32),
        ],
        compiler_params=pltpu.CompilerParams(
            dimension_semantics=("parallel", "arbitrary")),
    )(pre_acts, thresh, WdT)


def kernel(x, W_enc, W_dec, pre_bias, latent_bias):
    xc = x - pre_bias
    pre_acts = _encode(xc, W_enc, latent_bias)
    relu = jnp.maximum(pre_acts, 0.0)
    tv, ti = jax.lax.top_k(relu, K)
    thresh = tv[:, K - 1:K]
    sparse_code, recon_p = _scatter_decode(pre_acts, thresh, W_dec.T)
    recon = recon_p + pre_bias
    return (recon, sparse_code, pre_acts, tv, ti)


# speedup vs baseline: 2.6000x; 2.6000x over previous
import jax
import jax.numpy as jnp
from jax.experimental import pallas as pl
from jax.experimental.pallas import tpu as pltpu

N, D, H, K = 8192, 1024, 16384, 64
NB, HB = 2048, 1024
NB2, HB2 = 2048, 512


def _mm_kernel(x_ref, w_ref, b_ref, o_ref):
    o_ref[...] = jax.lax.dot_general(
        x_ref[...], w_ref[...], (((1,), (1,)), ((), ())),
        preferred_element_type=jnp.float32) + b_ref[...]


def _encode(xc, W_enc, latent_bias):
    return pl.pallas_call(
        _mm_kernel,
        grid=(N // NB, H // HB),
        in_specs=[
            pl.BlockSpec((NB, D), lambda i, j: (i, 0)),
            pl.BlockSpec((HB, D), lambda i, j: (j, 0)),
            pl.BlockSpec((1, HB), lambda i, j: (0, j)),
        ],
        out_specs=pl.BlockSpec((NB, HB), lambda i, j: (i, j)),
        out_shape=jax.ShapeDtypeStruct((N, H), jnp.float32),
    )(xc, W_enc, latent_bias.reshape(1, H))


NB3 = 128


def _topk_kernel(pa_ref, tv_ref, ti_ref):
    x = jnp.maximum(pa_ref[...], 0.0)
    iota = jax.lax.broadcasted_iota(jnp.int32, (NB3, H), 1)
    kiota = jax.lax.broadcasted_iota(jnp.int32, (NB3, K), 1)

    def body(k, carry):
        x, tva, tia = carry
        m = jnp.max(x, axis=1, keepdims=True)
        idx = jnp.min(jnp.where(x == m, iota, H), axis=1, keepdims=True)
        tva = jnp.where(kiota == k, m, tva)
        tia = jnp.where(kiota == k, idx, tia)
        return jnp.where(iota == idx, -1.0, x), tva, tia

    _, tva, tia = jax.lax.fori_loop(
        0, K, body,
        (x, jnp.zeros((NB3, K), jnp.float32), jnp.zeros((NB3, K), jnp.int32)))
    tv_ref[...] = tva
    ti_ref[...] = tia


def _topk(pre_acts):
    return pl.pallas_call(
        _topk_kernel,
        grid=(N // NB3,),
        in_specs=[pl.BlockSpec((NB3, H), lambda i: (i, 0))],
        out_specs=[
            pl.BlockSpec((NB3, K), lambda i: (i, 0)),
            pl.BlockSpec((NB3, K), lambda i: (i, 0)),
        ],
        out_shape=[
            jax.ShapeDtypeStruct((N, K), jnp.float32),
            jax.ShapeDtypeStruct((N, K), jnp.int32),
        ],
    )(pre_acts)


def _scatter_decode_kernel(pa_ref, t_ref, wdt_ref, sc_ref, rec_ref):
    j = pl.program_id(1)
    relu = jnp.maximum(pa_ref[...], 0.0)
    sc = jnp.where(relu >= t_ref[...], relu, 0.0)
    sc_ref[...] = sc

    @pl.when(j == 0)
    def _():
        rec_ref[...] = jnp.zeros_like(rec_ref)

    rec_ref[...] += jax.lax.dot_general(
        sc, wdt_ref[...], (((1,), (0,)), ((), ())),
        preferred_element_type=jnp.float32)


def _scatter_decode(pre_acts, thresh, WdT):
    return pl.pallas_call(
        _scatter_decode_kernel,
        grid=(N // NB2, H // HB2),
        in_specs=[
            pl.BlockSpec((NB2, HB2), lambda i, j: (i, j)),
            pl.BlockSpec((NB2, 1), lambda i, j: (i, 0)),
            pl.BlockSpec((HB2, D), lambda i, j: (j, 0)),
        ],
        out_specs=[
            pl.BlockSpec((NB2, HB2), lambda i, j: (i, j)),
            pl.BlockSpec((NB2, D), lambda i, j: (i, 0)),
        ],
        out_shape=[
            jax.ShapeDtypeStruct((N, H), jnp.float32),
            jax.ShapeDtypeStruct((N, D), jnp.float32),
        ],
        compiler_params=pltpu.CompilerParams(
            dimension_semantics=("parallel", "arbitrary")),
    )(pre_acts, thresh, WdT)


def kernel(x, W_enc, W_dec, pre_bias, latent_bias):
    xc = x - pre_bias
    pre_acts = _encode(xc, W_enc, latent_bias)
    tv, ti = _topk(pre_acts)
    thresh = tv[:, K - 1:K]
    sparse_code, recon_p = _scatter_decode(pre_acts, thresh, W_dec.T)
    recon = recon_p + pre_bias
    return (recon, sparse_code, pre_acts, tv, ti)
